# Initial kernel scaffold; baseline (speedup 1.0000x reference)
import jax
import jax.numpy as jnp
from jax.experimental import pallas as pl


def kernel(X, vertex, edges, degV, degE, H, alpha, beta, X0, W):
    raise NotImplementedError("write your pallas kernel here")


# speedup vs baseline: 6.1641x; 6.1641x over previous
import functools

import jax
import jax.numpy as jnp
from jax import lax
from jax.experimental import pallas as pl
from jax.experimental.pallas import tpu as pltpu
from jax.experimental.pallas import tpu_sc as plsc

_NC = 2
_NS = 16
_NW = _NC * _NS
_CHUNK = 128


def _round_up(x, m):
    return (x + m - 1) // m * m


def _gather_scatter_sum(src, gidx, sidx, dst_rows_pad, with_count):
    nnz = gidx.shape[0]
    d = src.shape[1]
    nchunks = nnz // _CHUNK
    maxiter = -(-nchunks // _NW)
    dst_per_sub = dst_rows_pad // _NS
    mesh = plsc.VectorSubcoreMesh(core_axis_name="c", subcore_axis_name="s")

    out_types = [jax.ShapeDtypeStruct((_NC * dst_rows_pad, d), jnp.float32)]
    scratch = [
        pltpu.VMEM((_CHUNK,), jnp.int32),
        pltpu.VMEM((_CHUNK,), jnp.int32),
        pltpu.VMEM((_CHUNK, d), jnp.float32),
        pltpu.VMEM_SHARED((dst_rows_pad, d), jnp.float32),
        pltpu.SemaphoreType.DMA,
    ]
    if with_count:
        out_types.append(
            jax.ShapeDtypeStruct((_NC * dst_rows_pad, 128), jnp.float32))
        scratch += [
            pltpu.VMEM((_CHUNK, 128), jnp.float32),
            pltpu.VMEM_SHARED((dst_rows_pad, 128), jnp.float32),
        ]

    zrow = jnp.zeros((dst_rows_pad, d), jnp.float32)
    zcnt = jnp.zeros((dst_rows_pad, 128), jnp.float32)
    ones = jnp.ones((_CHUNK, 128), jnp.float32)

    @functools.partial(
        pl.kernel,
        out_type=tuple(out_types),
        mesh=mesh,
        scratch_types=scratch,
    )
    def kern(*refs):
        if with_count:
            (src_hbm, gidx_hbm, sidx_hbm, zrow_hbm, zcnt_hbm, ones_hbm,
             out_hbm, cnt_hbm,
             gv, sv, rows, shared, sem, ones_v, shared_cnt) = refs
        else:
            (src_hbm, gidx_hbm, sidx_hbm, zrow_hbm,
             out_hbm,
             gv, sv, rows, shared, sem) = refs

        cid = lax.axis_index("c")
        sid = lax.axis_index("s")
        wid = sid * _NC + cid

        base0 = sid * dst_per_sub
        pltpu.sync_copy(zrow_hbm.at[pl.ds(base0, dst_per_sub)],
                        shared.at[pl.ds(base0, dst_per_sub)])
        if with_count:
            pltpu.sync_copy(zcnt_hbm.at[pl.ds(base0, dst_per_sub)],
                            shared_cnt.at[pl.ds(base0, dst_per_sub)])
            pltpu.sync_copy(ones_hbm, ones_v)

        plsc.subcore_barrier()

        @pl.loop(0, maxiter)
        def _(it):
            chunk = it * _NW + wid

            @pl.when(chunk < nchunks)
            def _():
                base = chunk * _CHUNK
                pltpu.sync_copy(gidx_hbm.at[pl.ds(base, _CHUNK)], gv)
                pltpu.sync_copy(sidx_hbm.at[pl.ds(base, _CHUNK)], sv)
                pltpu.async_copy(src_hbm.at[gv], rows, sem).wait()
                pltpu.sync_copy(rows, shared.at[sv], add=True)
                if with_count:
                    pltpu.sync_copy(ones_v, shared_cnt.at[sv], add=True)

        plsc.subcore_barrier()

        out_base = cid * dst_rows_pad + base0
        pltpu.sync_copy(shared.at[pl.ds(base0, dst_per_sub)],
                        out_hbm.at[pl.ds(out_base, dst_per_sub)])
        if with_count:
            pltpu.sync_copy(shared_cnt.at[pl.ds(base0, dst_per_sub)],
                            cnt_hbm.at[pl.ds(out_base, dst_per_sub)])

    if with_count:
        return kern(src, gidx, sidx, zrow, zcnt, ones)
    out = kern(src, gidx, sidx, zrow)
    if isinstance(out, (tuple, list)):
        out = out[0]
    return out, None


def _combine_edges(p0, p1, c0, c1, degE):
    m, d = p0.shape

    def body(p0_ref, p1_ref, c0_ref, c1_ref, degE_ref, out_ref):
        cnt = c0_ref[:, 0:1] + c1_ref[:, 0:1]
        sums = p0_ref[...] + p1_ref[...]
        out_ref[...] = sums / jnp.maximum(cnt, 1.0) * degE_ref[...]

    return pl.pallas_call(
        body,
        out_shape=jax.ShapeDtypeStruct((m, d), jnp.float32),
    )(p0, p1, c0, c1, degE)


def _vertex_epilogue(p0, p1, degV, X0, W, ab):
    n, d = p0.shape
    blk = 1000
    if n % blk != 0:
        blk = n
    grid = n // blk

    def body(ab_ref, p0_ref, p1_ref, degV_ref, X0_ref, W_ref, out_ref):
        alpha = ab_ref[0]
        beta = ab_ref[1]
        Xv = (p0_ref[...] + p1_ref[...]) * degV_ref[...]
        norm = jnp.sqrt(jnp.sum(Xv * Xv, axis=1, keepdims=True))
        Xn = Xv * jnp.where(norm > 0, 1.0 / norm, 0.0)
        Xi = (1.0 - alpha) * Xn + alpha * X0_ref[...]
        XiW = lax.dot_general(
            Xi, W_ref[...], (((1,), (1,)), ((), ())),
            preferred_element_type=jnp.float32,
            precision=lax.Precision.HIGHEST)
        out_ref[...] = (1.0 - beta) * Xi + beta * XiW

    return pl.pallas_call(
        body,
        grid=(grid,),
        in_specs=[
            pl.BlockSpec(memory_space=pltpu.SMEM),
            pl.BlockSpec((blk, d), lambda i: (i, 0)),
            pl.BlockSpec((blk, d), lambda i: (i, 0)),
            pl.BlockSpec((blk, 1), lambda i: (i, 0)),
            pl.BlockSpec((blk, d), lambda i: (i, 0)),
            pl.BlockSpec((d, d), lambda i: (0, 0)),
        ],
        out_specs=pl.BlockSpec((blk, d), lambda i: (i, 0)),
        out_shape=jax.ShapeDtypeStruct((n, d), jnp.float32),
    )(ab, p0, p1, degV, X0, W)


def kernel(X, vertex, edges, degV, degE, H, alpha, beta, X0, W):
    n, d = X.shape
    m = H.shape[1]
    nnz = vertex.shape[0]

    nnz_pad = _round_up(nnz, _CHUNK)
    m_pad = _round_up(m, 8 * _NS)
    n_pad = _round_up(n, 8 * _NS)
    if nnz_pad != nnz:
        if m_pad == m:
            m_pad += 8 * _NS
        if n_pad == n:
            n_pad += 8 * _NS
        pad = nnz_pad - nnz
        vertex_g = jnp.concatenate(
            [vertex, jnp.zeros((pad,), jnp.int32)])
        vertex_s = jnp.concatenate(
            [vertex, jnp.full((pad,), n_pad - 1, jnp.int32)])
        edges_g = jnp.concatenate(
            [edges, jnp.zeros((pad,), jnp.int32)])
        edges_s = jnp.concatenate(
            [edges, jnp.full((pad,), m_pad - 1, jnp.int32)])
    else:
        vertex_g = vertex_s = vertex
        edges_g = edges_s = edges

    ep, ec = _gather_scatter_sum(X, vertex_g, edges_s, m_pad, with_count=True)

    Xe = _combine_edges(ep[:m], ep[m_pad:m_pad + m],
                        ec[:m], ec[m_pad:m_pad + m], degE)

    vp, _ = _gather_scatter_sum(Xe, edges_g, vertex_s, n_pad,
                                with_count=False)

    ab = jnp.stack([alpha.astype(jnp.float32), beta.astype(jnp.float32)])
    return _vertex_epilogue(vp[:n], vp[n_pad:n_pad + n], degV, X0, W, ab)


# speedup vs baseline: 10.1172x; 1.6413x over previous
import functools

import jax
import jax.numpy as jnp
from jax import lax
from jax.experimental import pallas as pl
from jax.experimental.pallas import tpu as pltpu
from jax.experimental.pallas import tpu_sc as plsc

_NC = 2
_NS = 16
_NW = _NC * _NS
_CHUNK = 128


def _round_up(x, m):
    return (x + m - 1) // m * m


def _gather_scatter_sum(src, gidx, sidx, dst_rows_pad, with_count, chunk=_CHUNK):
    nnz = gidx.shape[0]
    d = src.shape[1]
    nchunks = nnz // chunk
    maxiter = -(-nchunks // _NW)
    dst_per_sub = dst_rows_pad // _NS
    mesh = plsc.VectorSubcoreMesh(core_axis_name="c", subcore_axis_name="s")

    out_types = [jax.ShapeDtypeStruct((_NC * dst_rows_pad, d), jnp.float32)]
    scratch = [
        pltpu.VMEM((chunk,), jnp.int32),
        pltpu.VMEM((chunk,), jnp.int32),
        pltpu.VMEM((chunk, d), jnp.float32),
        pltpu.VMEM((chunk,), jnp.int32),
        pltpu.VMEM((chunk,), jnp.int32),
        pltpu.VMEM((chunk, d), jnp.float32),
        pltpu.VMEM_SHARED((dst_rows_pad, d), jnp.float32),
        pltpu.SemaphoreType.DMA,
        pltpu.SemaphoreType.DMA,
        pltpu.SemaphoreType.DMA,
        pltpu.SemaphoreType.DMA,
    ]
    if with_count:
        out_types.append(
            jax.ShapeDtypeStruct((_NC * dst_rows_pad, 128), jnp.float32))
        scratch += [
            pltpu.VMEM((chunk, 128), jnp.float32),
            pltpu.VMEM_SHARED((dst_rows_pad, 128), jnp.float32),
        ]

    zrow = jnp.zeros((dst_rows_pad, d), jnp.float32)
    zcnt = jnp.zeros((dst_rows_pad, 128), jnp.float32)
    ones = jnp.ones((chunk, 128), jnp.float32)

    @functools.partial(
        pl.kernel,
        out_type=tuple(out_types),
        mesh=mesh,
        scratch_types=scratch,
    )
    def kern(*refs):
        if with_count:
            (src_hbm, gidx_hbm, sidx_hbm, zrow_hbm, zcnt_hbm, ones_hbm,
             out_hbm, cnt_hbm,
             gv0, sv0, rows0, gv1, sv1, rows1, shared,
             si0, si1, sg0, sg1, ones_v, shared_cnt) = refs
        else:
            (src_hbm, gidx_hbm, sidx_hbm, zrow_hbm,
             out_hbm,
             gv0, sv0, rows0, gv1, sv1, rows1, shared,
             si0, si1, sg0, sg1) = refs

        cid = lax.axis_index("c")
        sid = lax.axis_index("s")
        wid = sid * _NC + cid

        base0 = sid * dst_per_sub
        pltpu.sync_copy(zrow_hbm.at[pl.ds(base0, dst_per_sub)],
                        shared.at[pl.ds(base0, dst_per_sub)])
        if with_count:
            pltpu.sync_copy(zcnt_hbm.at[pl.ds(base0, dst_per_sub)],
                            shared_cnt.at[pl.ds(base0, dst_per_sub)])
            pltpu.sync_copy(ones_hbm, ones_v)

        plsc.subcore_barrier()

        def start_idx(k, gv, sv, si):
            g = k * _NW + wid

            @pl.when(g < nchunks)
            def _():
                base = g * chunk
                pltpu.async_copy(gidx_hbm.at[pl.ds(base, chunk)], gv, si)
                pltpu.async_copy(sidx_hbm.at[pl.ds(base, chunk)], sv, si)

        def start_gather(k, gv, sv, rows, si, sg):
            g = k * _NW + wid

            @pl.when(g < nchunks)
            def _():
                pltpu.make_async_copy(
                    gidx_hbm.at[pl.ds(0, chunk)], gv, si).wait()
                pltpu.make_async_copy(
                    sidx_hbm.at[pl.ds(0, chunk)], sv, si).wait()
                pltpu.async_copy(src_hbm.at[gv], rows, sg)

        def finish_scatter(k, gv, sv, rows, sg):
            g = k * _NW + wid

            @pl.when(g < nchunks)
            def _():
                pltpu.make_async_copy(src_hbm.at[gv], rows, sg).wait()
                pltpu.sync_copy(rows, shared.at[sv], add=True)
                if with_count:
                    pltpu.sync_copy(ones_v, shared_cnt.at[sv], add=True)

        start_idx(0, gv0, sv0, si0)
        start_idx(1, gv1, sv1, si1)
        start_gather(0, gv0, sv0, rows0, si0, sg0)

        @pl.loop(0, _round_up(maxiter, 2), step=2)
        def _(kk):
            start_gather(kk + 1, gv1, sv1, rows1, si1, sg1)
            finish_scatter(kk, gv0, sv0, rows0, sg0)
            start_idx(kk + 2, gv0, sv0, si0)
            start_gather(kk + 2, gv0, sv0, rows0, si0, sg0)
            finish_scatter(kk + 1, gv1, sv1, rows1, sg1)
            start_idx(kk + 3, gv1, sv1, si1)

        plsc.subcore_barrier()

        out_base = cid * dst_rows_pad + base0
        pltpu.sync_copy(shared.at[pl.ds(base0, dst_per_sub)],
                        out_hbm.at[pl.ds(out_base, dst_per_sub)])
        if with_count:
            pltpu.sync_copy(shared_cnt.at[pl.ds(base0, dst_per_sub)],
                            cnt_hbm.at[pl.ds(out_base, dst_per_sub)])

    if with_count:
        return kern(src, gidx, sidx, zrow, zcnt, ones)
    out = kern(src, gidx, sidx, zrow)
    if isinstance(out, (tuple, list)):
        out = out[0]
    return out, None


def _combine_edges(p0, p1, c0, c1, degE):
    m, d = p0.shape

    def body(p0_ref, p1_ref, c0_ref, c1_ref, degE_ref, out_ref):
        cnt = c0_ref[:, 0:1] + c1_ref[:, 0:1]
        sums = p0_ref[...] + p1_ref[...]
        out_ref[...] = sums / jnp.maximum(cnt, 1.0) * degE_ref[...]

    return pl.pallas_call(
        body,
        out_shape=jax.ShapeDtypeStruct((m, d), jnp.float32),
    )(p0, p1, c0, c1, degE)


def _vertex_epilogue(p0, p1, degV, X0, W, ab):
    n, d = p0.shape
    blk = 1000
    if n % blk != 0:
        blk = n
    grid = n // blk

    def body(ab_ref, p0_ref, p1_ref, degV_ref, X0_ref, W_ref, out_ref):
        alpha = ab_ref[0]
        beta = ab_ref[1]
        Xv = (p0_ref[...] + p1_ref[...]) * degV_ref[...]
        norm = jnp.sqrt(jnp.sum(Xv * Xv, axis=1, keepdims=True))
        Xn = Xv * jnp.where(norm > 0, 1.0 / norm, 0.0)
        Xi = (1.0 - alpha) * Xn + alpha * X0_ref[...]
        XiW = lax.dot_general(
            Xi, W_ref[...], (((1,), (1,)), ((), ())),
            preferred_element_type=jnp.float32,
            precision=lax.Precision.HIGHEST)
        out_ref[...] = (1.0 - beta) * Xi + beta * XiW

    return pl.pallas_call(
        body,
        grid=(grid,),
        in_specs=[
            pl.BlockSpec(memory_space=pltpu.SMEM),
            pl.BlockSpec((blk, d), lambda i: (i, 0)),
            pl.BlockSpec((blk, d), lambda i: (i, 0)),
            pl.BlockSpec((blk, 1), lambda i: (i, 0)),
            pl.BlockSpec((blk, d), lambda i: (i, 0)),
            pl.BlockSpec((d, d), lambda i: (0, 0)),
        ],
        out_specs=pl.BlockSpec((blk, d), lambda i: (i, 0)),
        out_shape=jax.ShapeDtypeStruct((n, d), jnp.float32),
    )(ab, p0, p1, degV, X0, W)


def _pad_idx(gidx, sidx, chunk, dump_row):
    nnz = gidx.shape[0]
    pad = _round_up(nnz, chunk) - nnz
    if pad == 0:
        return gidx, sidx
    return (jnp.concatenate([gidx, jnp.zeros((pad,), jnp.int32)]),
            jnp.concatenate([sidx, jnp.full((pad,), dump_row, jnp.int32)]))


def kernel(X, vertex, edges, degV, degE, H, alpha, beta, X0, W):
    n, d = X.shape
    m = H.shape[1]
    nnz = vertex.shape[0]

    chunk1 = 96
    chunk2 = _CHUNK

    m_pad = _round_up(m, 8 * _NS)
    n_pad = _round_up(n, 8 * _NS)
    if nnz % chunk1 and m_pad == m:
        m_pad += 8 * _NS
    if nnz % chunk2 and n_pad == n:
        n_pad += 8 * _NS

    vg1, es1 = _pad_idx(vertex, edges, chunk1, m_pad - 1)
    ep, ec = _gather_scatter_sum(X, vg1, es1, m_pad, with_count=True,
                                 chunk=chunk1)

    Xe = _combine_edges(ep[:m], ep[m_pad:m_pad + m],
                        ec[:m], ec[m_pad:m_pad + m], degE)

    eg2, vs2 = _pad_idx(edges, vertex, chunk2, n_pad - 1)
    vp, _ = _gather_scatter_sum(Xe, eg2, vs2, n_pad, with_count=False,
                                chunk=chunk2)

    ab = jnp.stack([alpha.astype(jnp.float32), beta.astype(jnp.float32)])
    return _vertex_epilogue(vp[:n], vp[n_pad:n_pad + n], degV, X0, W, ab)


# speedup vs baseline: 10.4207x; 1.0300x over previous
import functools

import jax
import jax.numpy as jnp
from jax import lax
from jax.experimental import pallas as pl
from jax.experimental.pallas import tpu as pltpu
from jax.experimental.pallas import tpu_sc as plsc

_NC = 2
_NS = 16
_NW = _NC * _NS
_CHUNK = 128


def _round_up(x, m):
    return (x + m - 1) // m * m


def _gather_scatter_sum(src, gidx, sidx, dst_rows_pad, with_count, chunk=_CHUNK):
    nnz = gidx.shape[0]
    d = src.shape[1]
    nchunks = nnz // chunk
    maxiter = -(-nchunks // _NW)
    dst_per_sub = dst_rows_pad // _NS
    mesh = plsc.VectorSubcoreMesh(core_axis_name="c", subcore_axis_name="s")

    out_types = [jax.ShapeDtypeStruct((_NC * dst_rows_pad, d), jnp.float32)]
    scratch = [
        pltpu.VMEM((chunk,), jnp.int32),
        pltpu.VMEM((chunk,), jnp.int32),
        pltpu.VMEM((chunk, d), jnp.float32),
        pltpu.VMEM((chunk,), jnp.int32),
        pltpu.VMEM((chunk,), jnp.int32),
        pltpu.VMEM((chunk, d), jnp.float32),
        pltpu.VMEM_SHARED((dst_rows_pad, d), jnp.float32),
        pltpu.SemaphoreType.DMA,
        pltpu.SemaphoreType.DMA,
        pltpu.SemaphoreType.DMA,
        pltpu.SemaphoreType.DMA,
        pltpu.SemaphoreType.DMA,
    ]
    if with_count:
        out_types.append(
            jax.ShapeDtypeStruct((_NC * dst_rows_pad, 128), jnp.float32))
        scratch += [
            pltpu.VMEM((chunk, 128), jnp.float32),
            pltpu.VMEM_SHARED((dst_rows_pad, 128), jnp.float32),
        ]

    zrow = jnp.zeros((dst_rows_pad, d), jnp.float32)
    zcnt = jnp.zeros((dst_rows_pad, 128), jnp.float32)
    ones = jnp.ones((chunk, 128), jnp.float32)

    @functools.partial(
        pl.kernel,
        out_type=tuple(out_types),
        mesh=mesh,
        scratch_types=scratch,
    )
    def kern(*refs):
        if with_count:
            (src_hbm, gidx_hbm, sidx_hbm, zrow_hbm, zcnt_hbm, ones_hbm,
             out_hbm, cnt_hbm,
             gv0, sv0, rows0, gv1, sv1, rows1, shared,
             si0, si1, sg0, sg1, ss, ones_v, shared_cnt) = refs
        else:
            (src_hbm, gidx_hbm, sidx_hbm, zrow_hbm,
             out_hbm,
             gv0, sv0, rows0, gv1, sv1, rows1, shared,
             si0, si1, sg0, sg1, ss) = refs

        cid = lax.axis_index("c")
        sid = lax.axis_index("s")
        wid = sid * _NC + cid

        base0 = sid * dst_per_sub
        pltpu.sync_copy(zrow_hbm.at[pl.ds(base0, dst_per_sub)],
                        shared.at[pl.ds(base0, dst_per_sub)])
        if with_count:
            pltpu.sync_copy(zcnt_hbm.at[pl.ds(base0, dst_per_sub)],
                            shared_cnt.at[pl.ds(base0, dst_per_sub)])
            pltpu.sync_copy(ones_hbm, ones_v)

        plsc.subcore_barrier()

        def start_idx(k, gv, sv, si):
            g = k * _NW + wid

            @pl.when(g < nchunks)
            def _():
                base = g * chunk
                pltpu.async_copy(gidx_hbm.at[pl.ds(base, chunk)], gv, si)
                pltpu.async_copy(sidx_hbm.at[pl.ds(base, chunk)], sv, si)

        def start_gather(k, gv, sv, rows, si, sg):
            g = k * _NW + wid

            @pl.when(g < nchunks)
            def _():
                pltpu.make_async_copy(
                    gidx_hbm.at[pl.ds(0, chunk)], gv, si).wait()
                pltpu.make_async_copy(
                    sidx_hbm.at[pl.ds(0, chunk)], sv, si).wait()
                pltpu.async_copy(src_hbm.at[gv], rows, sg)

        def finish_scatter(k, gv, sv, rows, sg):
            g = k * _NW + wid

            @pl.when(g < nchunks)
            def _():
                pltpu.make_async_copy(src_hbm.at[gv], rows, sg).wait()
                pltpu.async_copy(rows, shared.at[sv], ss, add=True)
                if with_count:
                    pltpu.async_copy(ones_v, shared_cnt.at[sv], ss, add=True)
                    pltpu.make_async_copy(ones_v, shared_cnt.at[sv], ss).wait()
                pltpu.make_async_copy(rows, shared.at[sv], ss).wait()

        start_idx(0, gv0, sv0, si0)
        start_idx(1, gv1, sv1, si1)
        start_gather(0, gv0, sv0, rows0, si0, sg0)

        @pl.loop(0, _round_up(maxiter, 2), step=2)
        def _(kk):
            start_gather(kk + 1, gv1, sv1, rows1, si1, sg1)
            finish_scatter(kk, gv0, sv0, rows0, sg0)
            start_idx(kk + 2, gv0, sv0, si0)
            start_gather(kk + 2, gv0, sv0, rows0, si0, sg0)
            finish_scatter(kk + 1, gv1, sv1, rows1, sg1)
            start_idx(kk + 3, gv1, sv1, si1)

        plsc.subcore_barrier()

        out_base = cid * dst_rows_pad + base0
        pltpu.sync_copy(shared.at[pl.ds(base0, dst_per_sub)],
                        out_hbm.at[pl.ds(out_base, dst_per_sub)])
        if with_count:
            pltpu.sync_copy(shared_cnt.at[pl.ds(base0, dst_per_sub)],
                            cnt_hbm.at[pl.ds(out_base, dst_per_sub)])

    if with_count:
        return kern(src, gidx, sidx, zrow, zcnt, ones)
    out = kern(src, gidx, sidx, zrow)
    if isinstance(out, (tuple, list)):
        out = out[0]
    return out, None


def _combine_edges(p0, p1, c0, c1, degE):
    m, d = p0.shape

    def body(p0_ref, p1_ref, c0_ref, c1_ref, degE_ref, out_ref):
        cnt = c0_ref[:, 0:1] + c1_ref[:, 0:1]
        sums = p0_ref[...] + p1_ref[...]
        out_ref[...] = sums / jnp.maximum(cnt, 1.0) * degE_ref[...]

    return pl.pallas_call(
        body,
        out_shape=jax.ShapeDtypeStruct((m, d), jnp.float32),
    )(p0, p1, c0, c1, degE)


def _vertex_epilogue(p0, p1, degV, X0, W, ab):
    n, d = p0.shape
    blk = 1000
    if n % blk != 0:
        blk = n
    grid = n // blk

    def body(ab_ref, p0_ref, p1_ref, degV_ref, X0_ref, W_ref, out_ref):
        alpha = ab_ref[0]
        beta = ab_ref[1]
        Xv = (p0_ref[...] + p1_ref[...]) * degV_ref[...]
        norm = jnp.sqrt(jnp.sum(Xv * Xv, axis=1, keepdims=True))
        Xn = Xv * jnp.where(norm > 0, 1.0 / norm, 0.0)
        Xi = (1.0 - alpha) * Xn + alpha * X0_ref[...]
        XiW = lax.dot_general(
            Xi, W_ref[...], (((1,), (1,)), ((), ())),
            preferred_element_type=jnp.float32,
            precision=lax.Precision.HIGHEST)
        out_ref[...] = (1.0 - beta) * Xi + beta * XiW

    return pl.pallas_call(
        body,
        grid=(grid,),
        in_specs=[
            pl.BlockSpec(memory_space=pltpu.SMEM),
            pl.BlockSpec((blk, d), lambda i: (i, 0)),
            pl.BlockSpec((blk, d), lambda i: (i, 0)),
            pl.BlockSpec((blk, 1), lambda i: (i, 0)),
            pl.BlockSpec((blk, d), lambda i: (i, 0)),
            pl.BlockSpec((d, d), lambda i: (0, 0)),
        ],
        out_specs=pl.BlockSpec((blk, d), lambda i: (i, 0)),
        out_shape=jax.ShapeDtypeStruct((n, d), jnp.float32),
    )(ab, p0, p1, degV, X0, W)


def _pad_idx(gidx, sidx, chunk, dump_row):
    nnz = gidx.shape[0]
    pad = _round_up(nnz, chunk) - nnz
    if pad == 0:
        return gidx, sidx
    return (jnp.concatenate([gidx, jnp.zeros((pad,), jnp.int32)]),
            jnp.concatenate([sidx, jnp.full((pad,), dump_row, jnp.int32)]))


def kernel(X, vertex, edges, degV, degE, H, alpha, beta, X0, W):
    n, d = X.shape
    m = H.shape[1]
    nnz = vertex.shape[0]

    chunk1 = 112
    chunk2 = _CHUNK

    m_pad = _round_up(m, 8 * _NS)
    n_pad = _round_up(n, 8 * _NS)
    if nnz % chunk1 and m_pad == m:
        m_pad += 8 * _NS
    if nnz % chunk2 and n_pad == n:
        n_pad += 8 * _NS

    vg1, es1 = _pad_idx(vertex, edges, chunk1, m_pad - 1)
    ep, ec = _gather_scatter_sum(X, vg1, es1, m_pad, with_count=True,
                                 chunk=chunk1)

    Xe = _combine_edges(ep[:m], ep[m_pad:m_pad + m],
                        ec[:m], ec[m_pad:m_pad + m], degE)

    eg2, vs2 = _pad_idx(edges, vertex, chunk2, n_pad - 1)
    vp, _ = _gather_scatter_sum(Xe, eg2, vs2, n_pad, with_count=False,
                                chunk=chunk2)

    ab = jnp.stack([alpha.astype(jnp.float32), beta.astype(jnp.float32)])
    return _vertex_epilogue(vp[:n], vp[n_pad:n_pad + n], degV, X0, W, ab)


# speedup vs baseline: 10.4316x; 1.0010x over previous
import functools

import jax
import jax.numpy as jnp
from jax import lax
from jax.experimental import pallas as pl
from jax.experimental.pallas import tpu as pltpu
from jax.experimental.pallas import tpu_sc as plsc

_NC = 2
_NS = 16
_NW = _NC * _NS
_CHUNK = 128


def _round_up(x, m):
    return (x + m - 1) // m * m


def _gather_scatter_sum(src, gidx, sidx, dst_rows_pad, with_count,
                        chunk=_CHUNK, nslot=3):
    nnz = gidx.shape[0]
    d = src.shape[1]
    nchunks = nnz // chunk
    maxiter = -(-nchunks // _NW)
    dst_per_sub = dst_rows_pad // _NS
    mesh = plsc.VectorSubcoreMesh(core_axis_name="c", subcore_axis_name="s")

    out_types = [jax.ShapeDtypeStruct((_NC * dst_rows_pad, d), jnp.float32)]
    scratch = []
    for _ in range(nslot):
        scratch += [
            pltpu.VMEM((chunk,), jnp.int32),
            pltpu.VMEM((chunk,), jnp.int32),
            pltpu.VMEM((chunk, d), jnp.float32),
        ]
    scratch.append(pltpu.VMEM_SHARED((dst_rows_pad, d), jnp.float32))
    scratch += [pltpu.SemaphoreType.DMA] * (2 * nslot + 1)
    if with_count:
        out_types.append(
            jax.ShapeDtypeStruct((_NC * dst_rows_pad, 128), jnp.float32))
        scratch += [
            pltpu.VMEM((chunk, 128), jnp.float32),
            pltpu.VMEM_SHARED((dst_rows_pad, 128), jnp.float32),
        ]

    zrow = jnp.zeros((dst_rows_pad, d), jnp.float32)
    zcnt = jnp.zeros((dst_rows_pad, 128), jnp.float32)
    ones = jnp.ones((chunk, 128), jnp.float32)

    @functools.partial(
        pl.kernel,
        out_type=tuple(out_types),
        mesh=mesh,
        scratch_types=scratch,
    )
    def kern(*refs):
        if with_count:
            (src_hbm, gidx_hbm, sidx_hbm, zrow_hbm, zcnt_hbm, ones_hbm,
             out_hbm, cnt_hbm) = refs[:8]
            scr = refs[8:]
            ones_v, shared_cnt = scr[-2:]
        else:
            (src_hbm, gidx_hbm, sidx_hbm, zrow_hbm, out_hbm) = refs[:5]
            scr = refs[5:]
        slots = [scr[3 * i:3 * i + 3] for i in range(nslot)]
        shared = scr[3 * nslot]
        sis = scr[3 * nslot + 1:3 * nslot + 1 + nslot]
        sgs = scr[3 * nslot + 1 + nslot:3 * nslot + 1 + 2 * nslot]
        ss = scr[3 * nslot + 1 + 2 * nslot]

        cid = lax.axis_index("c")
        sid = lax.axis_index("s")
        wid = sid * _NC + cid

        base0 = sid * dst_per_sub
        pltpu.sync_copy(zrow_hbm.at[pl.ds(base0, dst_per_sub)],
                        shared.at[pl.ds(base0, dst_per_sub)])
        if with_count:
            pltpu.sync_copy(zcnt_hbm.at[pl.ds(base0, dst_per_sub)],
                            shared_cnt.at[pl.ds(base0, dst_per_sub)])
            pltpu.sync_copy(ones_hbm, ones_v)

        plsc.subcore_barrier()

        def start_idx(k, j):
            gv, sv, _ = slots[j]
            g = k * _NW + wid

            @pl.when(g < nchunks)
            def _():
                base = g * chunk
                pltpu.async_copy(gidx_hbm.at[pl.ds(base, chunk)], gv, sis[j])
                pltpu.async_copy(sidx_hbm.at[pl.ds(base, chunk)], sv, sis[j])

        def start_gather(k, j):
            gv, sv, rows = slots[j]
            g = k * _NW + wid

            @pl.when(g < nchunks)
            def _():
                pltpu.make_async_copy(
                    gidx_hbm.at[pl.ds(0, chunk)], gv, sis[j]).wait()
                pltpu.make_async_copy(
                    sidx_hbm.at[pl.ds(0, chunk)], sv, sis[j]).wait()
                pltpu.async_copy(src_hbm.at[gv], rows, sgs[j])

        def finish_scatter(k, j):
            gv, sv, rows = slots[j]
            g = k * _NW + wid

            @pl.when(g < nchunks)
            def _():
                pltpu.make_async_copy(src_hbm.at[gv], rows, sgs[j]).wait()
                pltpu.async_copy(rows, shared.at[sv], ss, add=True)
                if with_count:
                    pltpu.async_copy(ones_v, shared_cnt.at[sv], ss, add=True)
                    pltpu.make_async_copy(ones_v, shared_cnt.at[sv], ss).wait()
                pltpu.make_async_copy(rows, shared.at[sv], ss).wait()

        for j in range(nslot):
            start_idx(j, j)
        for j in range(nslot - 1):
            start_gather(j, j)

        @pl.loop(0, _round_up(maxiter, nslot), step=nslot)
        def _(kk):
            for j in range(nslot):
                start_gather(kk + j + nslot - 1, (j + nslot - 1) % nslot)
                finish_scatter(kk + j, j)
                start_idx(kk + j + nslot, j)

        plsc.subcore_barrier()

        out_base = cid * dst_rows_pad + base0
        pltpu.sync_copy(shared.at[pl.ds(base0, dst_per_sub)],
                        out_hbm.at[pl.ds(out_base, dst_per_sub)])
        if with_count:
            pltpu.sync_copy(shared_cnt.at[pl.ds(base0, dst_per_sub)],
                            cnt_hbm.at[pl.ds(out_base, dst_per_sub)])

    if with_count:
        return kern(src, gidx, sidx, zrow, zcnt, ones)
    out = kern(src, gidx, sidx, zrow)
    if isinstance(out, (tuple, list)):
        out = out[0]
    return out, None


def _combine_edges(p0, p1, c0, c1, degE):
    m, d = p0.shape

    def body(p0_ref, p1_ref, c0_ref, c1_ref, degE_ref, out_ref):
        cnt = c0_ref[:, 0:1] + c1_ref[:, 0:1]
        sums = p0_ref[...] + p1_ref[...]
        out_ref[...] = sums / jnp.maximum(cnt, 1.0) * degE_ref[...]

    return pl.pallas_call(
        body,
        out_shape=jax.ShapeDtypeStruct((m, d), jnp.float32),
    )(p0, p1, c0, c1, degE)


def _vertex_epilogue(p0, p1, degV, X0, W, ab):
    n, d = p0.shape
    blk = 1000
    if n % blk != 0:
        blk = n
    grid = n // blk

    def body(ab_ref, p0_ref, p1_ref, degV_ref, X0_ref, W_ref, out_ref):
        alpha = ab_ref[0]
        beta = ab_ref[1]
        Xv = (p0_ref[...] + p1_ref[...]) * degV_ref[...]
        norm = jnp.sqrt(jnp.sum(Xv * Xv, axis=1, keepdims=True))
        Xn = Xv * jnp.where(norm > 0, 1.0 / norm, 0.0)
        Xi = (1.0 - alpha) * Xn + alpha * X0_ref[...]
        XiW = lax.dot_general(
            Xi, W_ref[...], (((1,), (1,)), ((), ())),
            preferred_element_type=jnp.float32,
            precision=lax.Precision.HIGHEST)
        out_ref[...] = (1.0 - beta) * Xi + beta * XiW

    return pl.pallas_call(
        body,
        grid=(grid,),
        in_specs=[
            pl.BlockSpec(memory_space=pltpu.SMEM),
            pl.BlockSpec((blk, d), lambda i: (i, 0)),
            pl.BlockSpec((blk, d), lambda i: (i, 0)),
            pl.BlockSpec((blk, 1), lambda i: (i, 0)),
            pl.BlockSpec((blk, d), lambda i: (i, 0)),
            pl.BlockSpec((d, d), lambda i: (0, 0)),
        ],
        out_specs=pl.BlockSpec((blk, d), lambda i: (i, 0)),
        out_shape=jax.ShapeDtypeStruct((n, d), jnp.float32),
    )(ab, p0, p1, degV, X0, W)


def _pad_idx(gidx, sidx, chunk, dump_row):
    nnz = gidx.shape[0]
    pad = _round_up(nnz, chunk) - nnz
    if pad == 0:
        return gidx, sidx
    return (jnp.concatenate([gidx, jnp.zeros((pad,), jnp.int32)]),
            jnp.concatenate([sidx, jnp.full((pad,), dump_row, jnp.int32)]))


def kernel(X, vertex, edges, degV, degE, H, alpha, beta, X0, W):
    n, d = X.shape
    m = H.shape[1]
    nnz = vertex.shape[0]

    chunk1 = 88
    chunk2 = _CHUNK

    m_pad = _round_up(m, 8 * _NS)
    n_pad = _round_up(n, 8 * _NS)
    if nnz % chunk1 and m_pad == m:
        m_pad += 8 * _NS
    if nnz % chunk2 and n_pad == n:
        n_pad += 8 * _NS

    vg1, es1 = _pad_idx(vertex, edges, chunk1, m_pad - 1)
    ep, ec = _gather_scatter_sum(X, vg1, es1, m_pad, with_count=True,
                                 chunk=chunk1)

    Xe = _combine_edges(ep[:m], ep[m_pad:m_pad + m],
                        ec[:m], ec[m_pad:m_pad + m], degE)

    eg2, vs2 = _pad_idx(edges, vertex, chunk2, n_pad - 1)
    vp, _ = _gather_scatter_sum(Xe, eg2, vs2, n_pad, with_count=False,
                                chunk=chunk2)

    ab = jnp.stack([alpha.astype(jnp.float32), beta.astype(jnp.float32)])
    return _vertex_epilogue(vp[:n], vp[n_pad:n_pad + n], degV, X0, W, ab)


# speedup vs baseline: 12.7765x; 1.2248x over previous
import dataclasses
import functools

import jax
import jax.numpy as jnp
from jax import lax
from jax.experimental import pallas as pl
from jax.experimental.pallas import tpu as pltpu
from jax.experimental.pallas import tpu_sc as plsc

_NC = 2
_NS = 16
_NW = _NC * _NS
_CHUNK = 128


def _round_up(x, m):
    return (x + m - 1) // m * m


def _gather_scatter_sum(src, gidx, sidx, dst_rows_pad, with_count,
                        chunk=_CHUNK, nslot=3):
    nnz = gidx.shape[0]
    d = src.shape[1]
    nchunks = nnz // chunk
    maxiter = -(-nchunks // _NW)
    dst_per_sub = dst_rows_pad // _NS
    mesh = plsc.VectorSubcoreMesh(core_axis_name="c", subcore_axis_name="s")

    out_types = [jax.ShapeDtypeStruct((_NC * dst_rows_pad, d), jnp.float32)]
    scratch = []
    for _ in range(nslot):
        scratch += [
            pltpu.VMEM((chunk,), jnp.int32),
            pltpu.VMEM((chunk,), jnp.int32),
            pltpu.VMEM((chunk, d), jnp.float32),
        ]
    scratch.append(pltpu.VMEM_SHARED((dst_rows_pad, d), jnp.float32))
    scratch += [pltpu.SemaphoreType.DMA] * (2 * nslot + 1)
    if with_count:
        out_types.append(
            jax.ShapeDtypeStruct((_NW, dst_rows_pad), jnp.float32))
        scratch.append(pltpu.VMEM((dst_rows_pad,), jnp.float32))

    zrow = jnp.zeros((dst_rows_pad, d), jnp.float32)
    zhist = jnp.zeros((dst_rows_pad,), jnp.float32)

    cp = pltpu.CompilerParams()
    if with_count and "needs_layout_passes" in (
            pltpu.CompilerParams.__dataclass_fields__):
        cp = dataclasses.replace(cp, needs_layout_passes=False)

    @functools.partial(
        pl.kernel,
        out_type=tuple(out_types),
        mesh=mesh,
        compiler_params=cp,
        scratch_types=scratch,
    )
    def kern(*refs):
        if with_count:
            (src_hbm, gidx_hbm, sidx_hbm, zrow_hbm, zhist_hbm,
             out_hbm, cnt_hbm) = refs[:7]
            scr = refs[7:]
            hist = scr[-1]
        else:
            (src_hbm, gidx_hbm, sidx_hbm, zrow_hbm, out_hbm) = refs[:5]
            scr = refs[5:]
        slots = [scr[3 * i:3 * i + 3] for i in range(nslot)]
        shared = scr[3 * nslot]
        sis = scr[3 * nslot + 1:3 * nslot + 1 + nslot]
        sgs = scr[3 * nslot + 1 + nslot:3 * nslot + 1 + 2 * nslot]
        ss = scr[3 * nslot + 1 + 2 * nslot]

        cid = lax.axis_index("c")
        sid = lax.axis_index("s")
        wid = sid * _NC + cid

        base0 = sid * dst_per_sub
        pltpu.sync_copy(zrow_hbm.at[pl.ds(base0, dst_per_sub)],
                        shared.at[pl.ds(base0, dst_per_sub)])
        if with_count:
            pltpu.sync_copy(zhist_hbm, hist)

        plsc.subcore_barrier()

        def start_idx(k, j):
            gv, sv, _ = slots[j]
            g = k * _NW + wid

            @pl.when(g < nchunks)
            def _():
                base = g * chunk
                pltpu.async_copy(gidx_hbm.at[pl.ds(base, chunk)], gv, sis[j])
                pltpu.async_copy(sidx_hbm.at[pl.ds(base, chunk)], sv, sis[j])

        def start_gather(k, j):
            gv, sv, rows = slots[j]
            g = k * _NW + wid

            @pl.when(g < nchunks)
            def _():
                pltpu.make_async_copy(
                    gidx_hbm.at[pl.ds(0, chunk)], gv, sis[j]).wait()
                pltpu.make_async_copy(
                    sidx_hbm.at[pl.ds(0, chunk)], sv, sis[j]).wait()
                pltpu.async_copy(src_hbm.at[gv], rows, sgs[j])

        def finish_scatter(k, j):
            gv, sv, rows = slots[j]
            g = k * _NW + wid

            @pl.when(g < nchunks)
            def _():
                pltpu.make_async_copy(src_hbm.at[gv], rows, sgs[j]).wait()
                pltpu.async_copy(rows, shared.at[sv], ss, add=True)
                if with_count:
                    for h in range(chunk // 16):
                        idx = sv[pl.ds(16 * h, 16)]
                        plsc.addupdate_scatter(
                            hist, [idx], jnp.full((16,), 1.0, jnp.float32))
                pltpu.make_async_copy(rows, shared.at[sv], ss).wait()

        for j in range(nslot):
            start_idx(j, j)
        for j in range(nslot - 1):
            start_gather(j, j)

        @pl.loop(0, _round_up(maxiter, nslot), step=nslot)
        def _(kk):
            for j in range(nslot):
                start_gather(kk + j + nslot - 1, (j + nslot - 1) % nslot)
                finish_scatter(kk + j, j)
                start_idx(kk + j + nslot, j)

        plsc.subcore_barrier()

        out_base = cid * dst_rows_pad + base0
        pltpu.sync_copy(shared.at[pl.ds(base0, dst_per_sub)],
                        out_hbm.at[pl.ds(out_base, dst_per_sub)])
        if with_count:
            pltpu.sync_copy(hist, cnt_hbm.at[wid])

    if with_count:
        return kern(src, gidx, sidx, zrow, zhist)
    out = kern(src, gidx, sidx, zrow)
    if isinstance(out, (tuple, list)):
        out = out[0]
    return out, None


def _combine_edges(ep, hists, degE_pad):
    m_pad, d = degE_pad.shape[0], ep.shape[1]
    blk = 512
    while m_pad % blk:
        blk //= 2
    nblk = m_pad // blk

    def body(p0_ref, p1_ref, h_ref, degE_ref, out_ref):
        ones = jnp.ones((_NW, 1), jnp.float32)
        cnt = lax.dot_general(
            h_ref[...], ones, (((0,), (0,)), ((), ())),
            preferred_element_type=jnp.float32,
            precision=lax.Precision.HIGHEST)
        sums = p0_ref[...] + p1_ref[...]
        out_ref[...] = sums / jnp.maximum(cnt, 1.0) * degE_ref[...]

    return pl.pallas_call(
        body,
        grid=(nblk,),
        in_specs=[
            pl.BlockSpec((blk, d), lambda i: (i, 0)),
            pl.BlockSpec((blk, d), lambda i: (i + nblk, 0)),
            pl.BlockSpec((_NW, blk), lambda i: (0, i)),
            pl.BlockSpec((blk, 1), lambda i: (i, 0)),
        ],
        out_specs=pl.BlockSpec((blk, d), lambda i: (i, 0)),
        out_shape=jax.ShapeDtypeStruct((m_pad, d), jnp.float32),
    )(ep, ep, hists, degE_pad)


def _vertex_epilogue(vp, degV_pad, X0_pad, W, ab):
    n_pad, d = X0_pad.shape
    blk = n_pad
    for cand in (632, 316, 158, 79, 512, 256, 128, 64, 8):
        if n_pad % cand == 0 and cand % 8 == 0:
            blk = cand
            break
    grid = n_pad // blk

    def body(ab_ref, p0_ref, p1_ref, degV_ref, X0_ref, W_ref, out_ref):
        alpha = ab_ref[0]
        beta = ab_ref[1]
        Xv = (p0_ref[...] + p1_ref[...]) * degV_ref[...]
        norm = jnp.sqrt(jnp.sum(Xv * Xv, axis=1, keepdims=True))
        Xn = Xv * jnp.where(norm > 0, 1.0 / norm, 0.0)
        Xi = (1.0 - alpha) * Xn + alpha * X0_ref[...]
        XiW = lax.dot_general(
            Xi, W_ref[...], (((1,), (1,)), ((), ())),
            preferred_element_type=jnp.float32,
            precision=lax.Precision.HIGHEST)
        out_ref[...] = (1.0 - beta) * Xi + beta * XiW

    return pl.pallas_call(
        body,
        grid=(grid,),
        in_specs=[
            pl.BlockSpec(memory_space=pltpu.SMEM),
            pl.BlockSpec((blk, d), lambda i: (i, 0)),
            pl.BlockSpec((blk, d), lambda i: (i + grid, 0)),
            pl.BlockSpec((blk, 1), lambda i: (i, 0)),
            pl.BlockSpec((blk, d), lambda i: (i, 0)),
            pl.BlockSpec((d, d), lambda i: (0, 0)),
        ],
        out_specs=pl.BlockSpec((blk, d), lambda i: (i, 0)),
        out_shape=jax.ShapeDtypeStruct((n_pad, d), jnp.float32),
    )(ab, vp, vp, degV_pad, X0_pad, W)


def _pad_idx(gidx, sidx, chunk, dump_row):
    nnz = gidx.shape[0]
    pad = _round_up(nnz, chunk) - nnz
    if pad == 0:
        return gidx, sidx
    return (jnp.concatenate([gidx, jnp.zeros((pad,), jnp.int32)]),
            jnp.concatenate([sidx, jnp.full((pad,), dump_row, jnp.int32)]))


def kernel(X, vertex, edges, degV, degE, H, alpha, beta, X0, W):
    n, d = X.shape
    m = H.shape[1]
    nnz = vertex.shape[0]

    chunk1 = _CHUNK
    chunk2 = _CHUNK

    m_pad = _round_up(m, 8 * _NS)
    n_pad = _round_up(n, 8 * _NS)
    if nnz % chunk1 and m_pad == m:
        m_pad += 8 * _NS
    if nnz % chunk2 and n_pad == n:
        n_pad += 8 * _NS

    vg1, es1 = _pad_idx(vertex, edges, chunk1, m_pad - 1)
    ep, hists = _gather_scatter_sum(X, vg1, es1, m_pad, with_count=True,
                                    chunk=chunk1)

    degE_pad = jnp.concatenate(
        [degE, jnp.zeros((m_pad - m, 1), jnp.float32)])
    Xe = _combine_edges(ep, hists, degE_pad)

    eg2, vs2 = _pad_idx(edges, vertex, chunk2, n_pad - 1)
    vp, _ = _gather_scatter_sum(Xe, eg2, vs2, n_pad, with_count=False,
                                chunk=chunk2)

    degV_pad = jnp.concatenate(
        [degV, jnp.zeros((n_pad - n, 1), jnp.float32)])
    X0_pad = jnp.concatenate(
        [X0, jnp.zeros((n_pad - n, d), jnp.float32)])
    ab = jnp.stack([alpha.astype(jnp.float32), beta.astype(jnp.float32)])
    return _vertex_epilogue(vp, degV_pad, X0_pad, W, ab)[:n]


# speedup vs baseline: 14.3800x; 1.1255x over previous
import dataclasses
import functools

import jax
import jax.numpy as jnp
from jax import lax
from jax.experimental import pallas as pl
from jax.experimental.pallas import tpu as pltpu
from jax.experimental.pallas import tpu_sc as plsc

_NC = 2
_NS = 16
_NW = _NC * _NS
_CHUNK = 128


def _round_up(x, m):
    return (x + m - 1) // m * m


def _gather_scatter_sum(src, gidx, sidx, dst_rows_pad, with_count,
                        chunk=_CHUNK, nslot=3):
    nnz = gidx.shape[0]
    d = src.shape[1]
    nchunks = nnz // chunk
    maxiter = -(-nchunks // _NW)
    dst_per_sub = dst_rows_pad // _NS
    mesh = plsc.VectorSubcoreMesh(core_axis_name="c", subcore_axis_name="s")

    out_types = [jax.ShapeDtypeStruct((_NC * dst_rows_pad, d), jnp.float32)]
    scratch = []
    for _ in range(nslot):
        scratch += [
            pltpu.VMEM((chunk,), jnp.int32),
            pltpu.VMEM((chunk,), jnp.int32),
            pltpu.VMEM((chunk,), jnp.int32),
            pltpu.VMEM((chunk, d), jnp.float32),
        ]
    scratch.append(pltpu.VMEM_SHARED((dst_rows_pad, d), jnp.float32))
    scratch += [pltpu.SemaphoreType.DMA] * (3 * nslot)
    if with_count:
        out_types.append(
            jax.ShapeDtypeStruct((_NW, dst_rows_pad), jnp.float32))
        scratch.append(pltpu.VMEM((dst_rows_pad,), jnp.float32))

    zrow = jnp.zeros((dst_rows_pad, d), jnp.float32)
    zhist = jnp.zeros((dst_rows_pad,), jnp.float32)

    cp = pltpu.CompilerParams()
    if with_count and "needs_layout_passes" in (
            pltpu.CompilerParams.__dataclass_fields__):
        cp = dataclasses.replace(cp, needs_layout_passes=False)

    @functools.partial(
        pl.kernel,
        out_type=tuple(out_types),
        mesh=mesh,
        compiler_params=cp,
        scratch_types=scratch,
    )
    def kern(*refs):
        if with_count:
            (src_hbm, gidx_hbm, sidx_hbm, zrow_hbm, zhist_hbm,
             out_hbm, cnt_hbm) = refs[:7]
            scr = refs[7:]
            hist = scr[-1]
        else:
            (src_hbm, gidx_hbm, sidx_hbm, zrow_hbm, out_hbm) = refs[:5]
            scr = refs[5:]
        slots = [scr[4 * i:4 * i + 4] for i in range(nslot)]
        shared = scr[4 * nslot]
        sems = scr[4 * nslot + 1:]
        sis = sems[:nslot]
        sgs = sems[nslot:2 * nslot]
        sss = sems[2 * nslot:3 * nslot]

        cid = lax.axis_index("c")
        sid = lax.axis_index("s")
        wid = sid * _NC + cid

        base0 = sid * dst_per_sub
        pltpu.sync_copy(zrow_hbm.at[pl.ds(base0, dst_per_sub)],
                        shared.at[pl.ds(base0, dst_per_sub)])
        if with_count:
            pltpu.sync_copy(zhist_hbm, hist)

        plsc.subcore_barrier()

        def start_idx(k, j):
            gv, sv, _, _ = slots[j]
            g = k * _NW + wid

            @pl.when(g < nchunks)
            def _():
                base = g * chunk
                pltpu.async_copy(gidx_hbm.at[pl.ds(base, chunk)], gv, sis[j])
                pltpu.async_copy(sidx_hbm.at[pl.ds(base, chunk)], sv, sis[j])

        def drain_scatter(k, j):
            _, _, svs, rows = slots[j]
            g = k * _NW + wid

            @pl.when(jnp.logical_and(k >= 0, g < nchunks))
            def _():
                pltpu.make_async_copy(rows, shared.at[svs], sss[j]).wait()

        def start_gather(k, j):
            gv, sv, svs, rows = slots[j]
            g = k * _NW + wid
            drain_scatter(k - nslot, j)

            @pl.when(g < nchunks)
            def _():
                pltpu.make_async_copy(
                    gidx_hbm.at[pl.ds(0, chunk)], gv, sis[j]).wait()
                pltpu.make_async_copy(
                    sidx_hbm.at[pl.ds(0, chunk)], sv, sis[j]).wait()
                pltpu.async_copy(src_hbm.at[gv], rows, sgs[j])

        def finish_scatter(k, j):
            gv, sv, svs, rows = slots[j]
            g = k * _NW + wid

            @pl.when(g < nchunks)
            def _():
                pltpu.make_async_copy(src_hbm.at[gv], rows, sgs[j]).wait()
                for h in range(chunk // 16):
                    svs[pl.ds(16 * h, 16)] = sv[pl.ds(16 * h, 16)]
                pltpu.async_copy(rows, shared.at[svs], sss[j], add=True)
                if with_count:
                    for h in range(chunk // 16):
                        idx = svs[pl.ds(16 * h, 16)]
                        plsc.addupdate_scatter(
                            hist, [idx], jnp.full((16,), 1.0, jnp.float32))

        for j in range(nslot):
            start_idx(j, j)
        for j in range(nslot - 1):
            start_gather(j, j)

        maxiter_r = _round_up(maxiter, nslot)

        @pl.loop(0, maxiter_r, step=nslot)
        def _(kk):
            for j in range(nslot):
                start_gather(kk + j + nslot - 1, (j + nslot - 1) % nslot)
                finish_scatter(kk + j, j)
                start_idx(kk + j + nslot, j)

        drain_scatter(maxiter_r - 1, (maxiter_r - 1) % nslot)

        plsc.subcore_barrier()

        out_base = cid * dst_rows_pad + base0
        pltpu.sync_copy(shared.at[pl.ds(base0, dst_per_sub)],
                        out_hbm.at[pl.ds(out_base, dst_per_sub)])
        if with_count:
            pltpu.sync_copy(hist, cnt_hbm.at[wid])

    if with_count:
        return kern(src, gidx, sidx, zrow, zhist)
    out = kern(src, gidx, sidx, zrow)
    if isinstance(out, (tuple, list)):
        out = out[0]
    return out, None


def _combine_edges(ep, hists, degE_pad):
    m_pad, d = degE_pad.shape[0], ep.shape[1]
    blk = 512
    while m_pad % blk:
        blk //= 2
    nblk = m_pad // blk

    def body(p0_ref, p1_ref, h_ref, degE_ref, out_ref):
        ones = jnp.ones((_NW, 1), jnp.float32)
        cnt = lax.dot_general(
            h_ref[...], ones, (((0,), (0,)), ((), ())),
            preferred_element_type=jnp.float32,
            precision=lax.Precision.HIGHEST)
        sums = p0_ref[...] + p1_ref[...]
        out_ref[...] = sums / jnp.maximum(cnt, 1.0) * degE_ref[...]

    return pl.pallas_call(
        body,
        grid=(nblk,),
        in_specs=[
            pl.BlockSpec((blk, d), lambda i: (i, 0)),
            pl.BlockSpec((blk, d), lambda i: (i + nblk, 0)),
            pl.BlockSpec((_NW, blk), lambda i: (0, i)),
            pl.BlockSpec((blk, 1), lambda i: (i, 0)),
        ],
        out_specs=pl.BlockSpec((blk, d), lambda i: (i, 0)),
        out_shape=jax.ShapeDtypeStruct((m_pad, d), jnp.float32),
    )(ep, ep, hists, degE_pad)


def _vertex_epilogue(vp, degV_pad, X0_pad, W, ab):
    n_pad, d = X0_pad.shape
    blk = n_pad
    for cand in (632, 316, 158, 79, 512, 256, 128, 64, 8):
        if n_pad % cand == 0 and cand % 8 == 0:
            blk = cand
            break
    grid = n_pad // blk

    def body(ab_ref, p0_ref, p1_ref, degV_ref, X0_ref, W_ref, out_ref):
        alpha = ab_ref[0]
        beta = ab_ref[1]
        Xv = (p0_ref[...] + p1_ref[...]) * degV_ref[...]
        norm = jnp.sqrt(jnp.sum(Xv * Xv, axis=1, keepdims=True))
        Xn = Xv * jnp.where(norm > 0, 1.0 / norm, 0.0)
        Xi = (1.0 - alpha) * Xn + alpha * X0_ref[...]
        XiW = lax.dot_general(
            Xi, W_ref[...], (((1,), (1,)), ((), ())),
            preferred_element_type=jnp.float32,
            precision=lax.Precision.HIGHEST)
        out_ref[...] = (1.0 - beta) * Xi + beta * XiW

    return pl.pallas_call(
        body,
        grid=(grid,),
        in_specs=[
            pl.BlockSpec(memory_space=pltpu.SMEM),
            pl.BlockSpec((blk, d), lambda i: (i, 0)),
            pl.BlockSpec((blk, d), lambda i: (i + grid, 0)),
            pl.BlockSpec((blk, 1), lambda i: (i, 0)),
            pl.BlockSpec((blk, d), lambda i: (i, 0)),
            pl.BlockSpec((d, d), lambda i: (0, 0)),
        ],
        out_specs=pl.BlockSpec((blk, d), lambda i: (i, 0)),
        out_shape=jax.ShapeDtypeStruct((n_pad, d), jnp.float32),
    )(ab, vp, vp, degV_pad, X0_pad, W)


def _pad_idx(gidx, sidx, chunk, dump_row):
    nnz = gidx.shape[0]
    pad = _round_up(nnz, chunk) - nnz
    if pad == 0:
        return gidx, sidx
    return (jnp.concatenate([gidx, jnp.zeros((pad,), jnp.int32)]),
            jnp.concatenate([sidx, jnp.full((pad,), dump_row, jnp.int32)]))


def kernel(X, vertex, edges, degV, degE, H, alpha, beta, X0, W):
    n, d = X.shape
    m = H.shape[1]
    nnz = vertex.shape[0]

    chunk1 = _CHUNK
    chunk2 = _CHUNK

    m_pad = _round_up(m, 8 * _NS)
    n_pad = _round_up(n, 8 * _NS)
    if nnz % chunk1 and m_pad == m:
        m_pad += 8 * _NS
    if nnz % chunk2 and n_pad == n:
        n_pad += 8 * _NS

    vg1, es1 = _pad_idx(vertex, edges, chunk1, m_pad - 1)
    ep, hists = _gather_scatter_sum(X, vg1, es1, m_pad, with_count=True,
                                    chunk=chunk1)

    degE_pad = jnp.concatenate(
        [degE, jnp.zeros((m_pad - m, 1), jnp.float32)])
    Xe = _combine_edges(ep, hists, degE_pad)

    eg2, vs2 = _pad_idx(edges, vertex, chunk2, n_pad - 1)
    vp, _ = _gather_scatter_sum(Xe, eg2, vs2, n_pad, with_count=False,
                                chunk=chunk2, nslot=2)

    degV_pad = jnp.concatenate(
        [degV, jnp.zeros((n_pad - n, 1), jnp.float32)])
    X0_pad = jnp.concatenate(
        [X0, jnp.zeros((n_pad - n, d), jnp.float32)])
    ab = jnp.stack([alpha.astype(jnp.float32), beta.astype(jnp.float32)])
    return _vertex_epilogue(vp, degV_pad, X0_pad, W, ab)[:n]
